# baseline (speedup 1.0000x reference)
import functools

import jax
import jax.numpy as jnp
from jax import lax
from jax.experimental import pallas as pl
from jax.experimental.pallas import tpu as pltpu
from jax.experimental.pallas import tpu_sc as plsc

NC = 2
NS = 16
L = 16
CHUNK = 256
SUB = CHUNK // 128


def _leaky(t):
    return jnp.maximum(t, 0.01 * t)


def _stage_a_body(x_ref, w1_ref, b1_ref, y_ref, skip_ref):
    y = lax.dot_general(x_ref[...], w1_ref[...], (((1,), (1,)), ((), ())),
                        preferred_element_type=jnp.float32)
    y_ref[...] = jnp.concatenate(
        [y, jnp.zeros((y.shape[0], 96), jnp.float32)], axis=1)
    skip_ref[...] = _leaky(y + b1_ref[...])


def _stage_c_body(s1a_ref, s1b_ref, d0_ref, d1_ref, skip_ref, w3_ref,
                  b3_ref, o0_ref, skip2_ref):
    cnt = d0_ref[:, :1] + d1_ref[:, :1]
    inv = 1.0 / jnp.maximum(cnt, 1.0)
    out1 = (jnp.concatenate([s1a_ref[:, :16], s1b_ref[:, :16]], axis=1)
            * inv + skip_ref[...])
    y2 = lax.dot_general(out1, w3_ref[...], (((1,), (1,)), ((), ())),
                         preferred_element_type=jnp.float32)
    o0_ref[...] = jnp.concatenate(
        [y2, jnp.zeros((y2.shape[0], 64), jnp.float32)], axis=1)
    skip2_ref[...] = _leaky(y2 + b3_ref[...])


def _stage_e_body(n_nodes, grid_n, s20_ref, s21_ref, s22_ref, s23_ref,
                  d0_ref, d1_ref, skip2_ref, w7_ref, b7_ref, out_ref, acc_ref):
    i = pl.program_id(0)

    @pl.when(i == 0)
    def _():
        acc_ref[...] = jnp.zeros_like(acc_ref)

    cnt = d0_ref[:, :1] + d1_ref[:, :1]
    inv = 1.0 / jnp.maximum(cnt, 1.0)
    out2 = (jnp.concatenate([s20_ref[:, :16], s21_ref[:, :16],
                             s22_ref[:, :16], s23_ref[:, :16]], axis=1)
            * inv + skip2_ref[...])
    r = out2.shape[0]
    acc_ref[...] += jnp.sum(out2.reshape(r // 8, 8, 64), axis=0)

    @pl.when(i == grid_n - 1)
    def _():
        pooled = jnp.sum(acc_ref[...], axis=0, keepdims=True) * (1.0 / n_nodes)
        logits = lax.dot_general(pooled, w7_ref[...], (((1,), (1,)), ((), ())),
                                 preferred_element_type=jnp.float32) + b7_ref[...]
        m = jnp.max(logits, axis=1, keepdims=True)
        out_ref[...] = (logits - m) - jnp.log(
            jnp.sum(jnp.exp(logits - m), axis=1, keepdims=True))


def _make_pass_kernel(num_groups, npad, epad, with_deg):
    gpc = num_groups // NC
    rows_pt = npad // NS
    ept = epad // NS
    chunks = ept // CHUNK
    ept2 = epad // (NC * NS)
    chunks2 = ept2 // CHUNK
    mesh = plsc.VectorSubcoreMesh(core_axis_name="c", subcore_axis_name="s",
                                  num_cores=NC, num_subcores=NS)
    nout = num_groups + (2 if with_deg else 0)
    out_type = ([jax.ShapeDtypeStruct((npad, 16), jnp.float32)
                 for _ in range(num_groups)]
                + [jax.ShapeDtypeStruct((npad, 128), jnp.float32)
                   for _ in range(nout)])
    scratch = [
        [pltpu.VMEM((SUB, 128), jnp.int32) for _ in range(4)],
        [pltpu.VMEM((SUB, 128), jnp.int32) for _ in range(4)],
        [pltpu.VMEM((CHUNK,), jnp.float32) for _ in range(4)],
        [pltpu.VMEM((CHUNK, L), jnp.float32) for _ in range(4)],
        [pltpu.VMEM((CHUNK, L), jnp.float32) for _ in range(2)],
        pltpu.VMEM((L,), jnp.float32),
        pltpu.MemorySpace.VMEM_SHARED((npad, 16), jnp.float32),
        [pltpu.SemaphoreType.DMA for _ in range(4)],
        [pltpu.SemaphoreType.DMA for _ in range(4)],
        [pltpu.SemaphoreType.DMA for _ in range(4)],
        [pltpu.SemaphoreType.DMA for _ in range(4)],
        [pltpu.SemaphoreType.DMA for _ in range(2)],
    ]

    assert chunks % 4 == 0 and chunks2 % 4 == 0
    kout = chunks // 4
    kout2 = chunks2 // 4

    @functools.partial(pl.kernel, out_type=out_type, mesh=mesh,
                       scratch_types=scratch,
                       compiler_params=pltpu.CompilerParams(
                           use_tc_tiling_on_sc=False))
    def pass_kernel(*refs):
        y128_hbm, col2_hbm, row2_hbm, wflat_hbm, b_hbm, zeros_hbm = refs[:6]
        nin = 7 if with_deg else 6
        if with_deg:
            ones_hbm = refs[6]
        tabs = refs[nin:nin + num_groups]
        outs = refs[nin + num_groups:nin + 2 * num_groups]
        if with_deg:
            d_hbm = refs[nin + 2 * num_groups:nin + 2 * num_groups + 2]
            rest = refs[nin + 2 * num_groups + 2:]
        else:
            d_hbm = None
            rest = refs[nin + 2 * num_groups:]
        (colv, rowv, wv, gath, val, bvec_s, acc,
         sem_ac, sem_ar, sem_aw, sem_g, sem_s) = rest
        c = lax.axis_index("c")
        s = lax.axis_index("s")
        r0 = s * rows_pt

        for core in range(NC):
            @pl.when(c == core)
            def _(core=core):
                for gl in range(gpc):
                    gidx = core * gpc + gl
                    pltpu.sync_copy(
                        y128_hbm.at[pl.ds(r0, rows_pt),
                                    pl.ds(gidx * 16, 16)],
                        tabs[gidx].at[pl.ds(r0, rows_pt)])
        plsc.subcore_barrier()

        def zero_acc():
            pltpu.sync_copy(zeros_hbm.at[pl.ds(r0, rows_pt)],
                            acc.at[pl.ds(r0, rows_pt)])

        def flush_acc(out):
            pltpu.sync_copy(acc.at[pl.ds(r0, rows_pt)],
                            out.at[pl.ds(r0, rows_pt), pl.ds(0, 16)])

        if with_deg:
            wid = c * NS + s

            def ar2(k, u):
                rb = wid * (ept2 // 128) + k * SUB
                return pltpu.make_async_copy(row2_hbm.at[pl.ds(rb, SUB)],
                                             rowv[u % 4], sem_ar[u % 4])

            def s2_copies(k, u):
                return [
                    pltpu.make_async_copy(val[1].at[pl.ds(0, 128)],
                                          acc.at[rowv[u % 4].at[j]],
                                          sem_s[u % 2])
                    for j in range(SUB)
                ]

            zero_acc()
            pltpu.sync_copy(ones_hbm, val[1].at[pl.ds(0, 128)])
            plsc.subcore_barrier()
            ar2(0, 0).start()
            ar2(1, 1).start()

            def deg_body(k0, carry):
                for u in range(4):
                    k = k0 * 4 + u

                    def drain(k=k, u=u):
                        for cp in s2_copies(k - 2, u + 2):
                            cp.wait()

                    if u < 2:
                        pl.when(k0 > 0)(drain)
                    else:
                        drain()

                    def prefetch(k=k, u=u):
                        ar2(k + 2, u + 2).start()

                    if u < 2:
                        prefetch()
                    else:
                        pl.when(k0 < kout2 - 1)(prefetch)
                    ar2(k, u).wait()
                    for cp in s2_copies(k, u):
                        cp.start(add=True)
                return carry

            lax.fori_loop(0, kout2, deg_body, 0)
            for cp in s2_copies(chunks2 - 2, 2):
                cp.wait()
            for cp in s2_copies(chunks2 - 1, 3):
                cp.wait()
            plsc.subcore_barrier()
            for core in range(NC):
                @pl.when(c == core)
                def _(core=core):
                    flush_acc(d_hbm[core])
            plsc.subcore_barrier()

        def run_group(tab, out, gidx):
            def ac(k, u):
                rb = s * (ept // 128) + k * SUB
                return pltpu.make_async_copy(col2_hbm.at[pl.ds(rb, SUB)],
                                             colv[u % 4], sem_ac[u % 4])

            def ar(k, u):
                rb = s * (ept // 128) + k * SUB
                return pltpu.make_async_copy(row2_hbm.at[pl.ds(rb, SUB)],
                                             rowv[u % 4], sem_ar[u % 4])

            def aw(k, u):
                eb = s * ept + k * CHUNK
                return pltpu.make_async_copy(wflat_hbm.at[pl.ds(eb, CHUNK)],
                                             wv[u % 4], sem_aw[u % 4])

            def g_copies(k, u):
                return [
                    pltpu.make_async_copy(tab.at[colv[u % 4].at[j]],
                                          gath[u % 4].at[pl.ds(j * 128, 128)],
                                          sem_g[u % 4])
                    for j in range(SUB)
                ]

            def s_copies(k, u):
                return [
                    pltpu.make_async_copy(val[u % 2].at[pl.ds(j * 128, 128)],
                                          acc.at[rowv[u % 4].at[j]],
                                          sem_s[u % 2])
                    for j in range(SUB)
                ]

            zero_acc()
            pltpu.sync_copy(b_hbm.at[gidx], bvec_s)
            plsc.subcore_barrier()

            for j in range(4):
                ac(j, j).start()
            for j in range(2):
                ar(j, j).start()
                aw(j, j).start()
            ac(0, 0).wait()
            for cp in g_copies(0, 0):
                cp.start()
            ac(1, 1).wait()
            for cp in g_copies(1, 1):
                cp.start()

            def outer_body(k0, carry):
                for u in range(4):
                    k = k0 * 4 + u

                    def drain_s(k=k, u=u):
                        for cp in s_copies(k - 2, u + 2):
                            cp.wait()

                    if u < 2:
                        pl.when(k0 > 0)(drain_s)
                    else:
                        drain_s()

                    def prefetch_rw(k=k, u=u):
                        ar(k + 2, u + 2).start()
                        aw(k + 2, u + 2).start()

                    if u < 2:
                        prefetch_rw()
                    else:
                        pl.when(k0 < kout - 1)(prefetch_rw)

                    for cp in g_copies(k, u):
                        cp.wait()

                    def prefetch_c(k=k, u=u):
                        ac(k + 4, u).start()

                    pl.when(k0 < kout - 1)(prefetch_c)

                    def start_g(k=k, u=u):
                        ac(k + 2, u + 2).wait()
                        for cp in g_copies(k + 2, u + 2):
                            cp.start()

                    if u < 2:
                        start_g()
                    else:
                        pl.when(k0 < kout - 1)(start_g)

                    aw(k, u).wait()
                    bvec = bvec_s[...]
                    gbuf = gath[u % 4]
                    vbuf = val[u % 2]
                    wbuf = wv[u % 4]

                    def grp_body(j, carry2):
                        wg = wbuf[pl.ds(j * L, L)]
                        for jj in range(L):
                            e = j * L + jj
                            wj = jnp.broadcast_to(
                                lax.slice(wg, (jj,), (jj + 1,)), (L,))
                            t = gbuf[e, :] * wj + bvec
                            vbuf[e, :] = jnp.maximum(t, 0.01 * t)
                        return carry2

                    lax.fori_loop(0, CHUNK // L, grp_body, 0)
                    ar(k, u).wait()
                    for cp in s_copies(k, u):
                        cp.start(add=True)
                return carry

            lax.fori_loop(0, kout, outer_body, 0)
            for cp in s_copies(chunks - 2, 2):
                cp.wait()
            for cp in s_copies(chunks - 1, 3):
                cp.wait()
            plsc.subcore_barrier()
            flush_acc(out)
            plsc.subcore_barrier()

        for core in range(NC):
            @pl.when(c == core)
            def _(core=core):
                for gl in range(gpc):
                    gidx = core * gpc + gl
                    run_group(tabs[gidx], outs[gidx], gidx)

    return pass_kernel


def kernel(x, edge, weight, W1, b1, W3, b3, W7, b7):
    n, _ = x.shape
    e = edge.shape[1]
    row = edge[0]
    col = edge[1]

    npad = ((n + 1 + 127) // 128) * 128
    estep = NC * NS * CHUNK
    epad = ((e + estep - 1) // estep) * estep
    pad = epad - e
    colp = jnp.concatenate([col, jnp.zeros((pad,), jnp.int32)])
    rowp = jnp.concatenate([row, jnp.full((pad,), n, jnp.int32)])
    wp = jnp.concatenate([weight, jnp.zeros((pad,), jnp.float32)])
    col2 = colp.reshape(-1, 128)
    row2 = rowp.reshape(-1, 128)
    zeros_hbm = jnp.zeros((npad, 16), jnp.float32)
    ones_hbm = jnp.ones((128, 16), jnp.float32)

    blk = 2000
    grid_n = n // blk
    wspec = pl.BlockSpec((blk, 128), lambda i: (i, 0))

    y1_128, skip1 = pl.pallas_call(
        _stage_a_body,
        grid=(grid_n,),
        in_specs=[
            pl.BlockSpec((blk, x.shape[1]), lambda i: (i, 0)),
            pl.BlockSpec(W1.shape, lambda i: (0, 0)),
            pl.BlockSpec((1, 32), lambda i: (0, 0)),
        ],
        out_specs=[wspec, pl.BlockSpec((blk, 32), lambda i: (i, 0))],
        out_shape=[
            jax.ShapeDtypeStruct((npad, 128), jnp.float32),
            jax.ShapeDtypeStruct((n, 32), jnp.float32),
        ],
    )(x, W1, b1.reshape(1, 32))

    _, _, s1a, s1b, d0, d1 = _make_pass_kernel(2, npad, epad, True)(
        y1_128, col2, row2, wp, b1.reshape(2, 16), zeros_hbm, ones_hbm)

    y2_128, skip2 = pl.pallas_call(
        _stage_c_body,
        grid=(grid_n,),
        in_specs=[wspec, wspec, wspec, wspec,
                  pl.BlockSpec((blk, 32), lambda i: (i, 0)),
                  pl.BlockSpec(W3.shape, lambda i: (0, 0)),
                  pl.BlockSpec((1, 64), lambda i: (0, 0))],
        out_specs=[wspec, pl.BlockSpec((blk, 64), lambda i: (i, 0))],
        out_shape=[jax.ShapeDtypeStruct((npad, 128), jnp.float32),
                   jax.ShapeDtypeStruct((n, 64), jnp.float32)],
    )(s1a, s1b, d0, d1, skip1, W3, b3.reshape(1, 64))

    _, _, _, _, s20, s21, s22, s23 = _make_pass_kernel(4, npad, epad, False)(
        y2_128, col2, row2, wp, b3.reshape(4, 16), zeros_hbm)

    out = pl.pallas_call(
        functools.partial(_stage_e_body, n, grid_n),
        grid=(grid_n,),
        in_specs=[wspec, wspec, wspec, wspec, wspec, wspec,
                  pl.BlockSpec((blk, 64), lambda i: (i, 0)),
                  pl.BlockSpec(W7.shape, lambda i: (0, 0)),
                  pl.BlockSpec((1, 2), lambda i: (0, 0))],
        out_specs=pl.BlockSpec((1, 2), lambda i: (0, 0)),
        out_shape=jax.ShapeDtypeStruct((1, 2), jnp.float32),
        scratch_shapes=[pltpu.VMEM((8, 64), jnp.float32)],
    )(s20, s21, s22, s23, d0, d1, skip2, W7, b7.reshape(1, 2))
    return out


# speedup vs baseline: 2.2878x; 2.2878x over previous
import functools

import jax
import jax.numpy as jnp
from jax import lax
from jax.experimental import pallas as pl
from jax.experimental.pallas import tpu as pltpu
from jax.experimental.pallas import tpu_sc as plsc

NC = 2
NS = 16
L = 16
CHUNK = 256
SUB = CHUNK // 128


def _leaky(t):
    return jnp.maximum(t, 0.01 * t)


def _stage_a_body(x_ref, w1_ref, b1_ref, ya_ref, yb_ref, skip_ref):
    y = lax.dot_general(x_ref[...], w1_ref[...], (((1,), (1,)), ((), ())),
                        preferred_element_type=jnp.float32)
    ya_ref[...] = y[:, :16]
    yb_ref[...] = y[:, 16:]
    skip_ref[...] = _leaky(y + b1_ref[...])


def _stage_c_body(s1a_ref, s1b_ref, d0_ref, d1_ref, skip_ref, w3_ref,
                  b3_ref, o0_ref, o1_ref, o2_ref, o3_ref, skip2_ref):
    cnt = d0_ref[:, :1] + d1_ref[:, :1]
    inv = 1.0 / jnp.maximum(cnt, 1.0)
    out1 = (jnp.concatenate([s1a_ref[:, :16], s1b_ref[:, :16]], axis=1)
            * inv + skip_ref[...])
    y2 = lax.dot_general(out1, w3_ref[...], (((1,), (1,)), ((), ())),
                         preferred_element_type=jnp.float32)
    o0_ref[...] = y2[:, 0:16]
    o1_ref[...] = y2[:, 16:32]
    o2_ref[...] = y2[:, 32:48]
    o3_ref[...] = y2[:, 48:64]
    skip2_ref[...] = _leaky(y2 + b3_ref[...])


def _stage_e_body(n_nodes, grid_n, s20_ref, s21_ref, s22_ref, s23_ref,
                  d0_ref, d1_ref, skip2_ref, w7_ref, b7_ref, out_ref, acc_ref):
    i = pl.program_id(0)

    @pl.when(i == 0)
    def _():
        acc_ref[...] = jnp.zeros_like(acc_ref)

    cnt = d0_ref[:, :1] + d1_ref[:, :1]
    inv = 1.0 / jnp.maximum(cnt, 1.0)
    out2 = (jnp.concatenate([s20_ref[:, :16], s21_ref[:, :16],
                             s22_ref[:, :16], s23_ref[:, :16]], axis=1)
            * inv + skip2_ref[...])
    r = out2.shape[0]
    acc_ref[...] += jnp.sum(out2.reshape(r // 8, 8, 64), axis=0)

    @pl.when(i == grid_n - 1)
    def _():
        pooled = jnp.sum(acc_ref[...], axis=0, keepdims=True) * (1.0 / n_nodes)
        logits = lax.dot_general(pooled, w7_ref[...], (((1,), (1,)), ((), ())),
                                 preferred_element_type=jnp.float32) + b7_ref[...]
        m = jnp.max(logits, axis=1, keepdims=True)
        out_ref[...] = (logits - m) - jnp.log(
            jnp.sum(jnp.exp(logits - m), axis=1, keepdims=True))


def _make_pass_kernel(num_groups, npad, epad, with_deg):
    gpc = num_groups // NC
    rows_pt = npad // NS
    ept = epad // NS
    chunks = ept // CHUNK
    ept2 = epad // (NC * NS)
    chunks2 = ept2 // CHUNK
    mesh = plsc.VectorSubcoreMesh(core_axis_name="c", subcore_axis_name="s",
                                  num_cores=NC, num_subcores=NS)
    nout = num_groups + (2 if with_deg else 0)
    out_type = [jax.ShapeDtypeStruct((npad, 128), jnp.float32)
                for _ in range(nout)]
    scratch = [
        [pltpu.VMEM((SUB, 128), jnp.int32) for _ in range(4)],
        [pltpu.VMEM((SUB, 128), jnp.int32) for _ in range(4)],
        [pltpu.VMEM((CHUNK,), jnp.float32) for _ in range(4)],
        [pltpu.VMEM((CHUNK, L), jnp.float32) for _ in range(4)],
        [pltpu.VMEM((CHUNK, L), jnp.float32) for _ in range(2)],
        pltpu.VMEM((L,), jnp.float32),
        pltpu.MemorySpace.VMEM_SHARED((npad, 16), jnp.float32),
        [pltpu.SemaphoreType.DMA for _ in range(4)],
        [pltpu.SemaphoreType.DMA for _ in range(4)],
        [pltpu.SemaphoreType.DMA for _ in range(4)],
        [pltpu.SemaphoreType.DMA for _ in range(4)],
        [pltpu.SemaphoreType.DMA for _ in range(2)],
    ]

    assert chunks % 4 == 0 and chunks2 % 4 == 0
    kout = chunks // 4
    kout2 = chunks2 // 4

    @functools.partial(pl.kernel, out_type=out_type, mesh=mesh,
                       scratch_types=scratch,
                       compiler_params=pltpu.CompilerParams(
                           use_tc_tiling_on_sc=False))
    def pass_kernel(*refs):
        tabs = refs[:num_groups]
        col2_hbm, row2_hbm, wflat_hbm, b_hbm, zeros_hbm = refs[num_groups:num_groups + 5]
        if with_deg:
            ones_hbm = refs[num_groups + 5]
            outs = refs[num_groups + 6:2 * num_groups + 6]
            d_hbm = refs[2 * num_groups + 6:2 * num_groups + 8]
            rest = refs[2 * num_groups + 8:]
        else:
            outs = refs[num_groups + 5:2 * num_groups + 5]
            d_hbm = None
            rest = refs[2 * num_groups + 5:]
        (colv, rowv, wv, gath, val, bvec_s, acc,
         sem_ac, sem_ar, sem_aw, sem_g, sem_s) = rest
        c = lax.axis_index("c")
        s = lax.axis_index("s")
        r0 = s * rows_pt

        def zero_acc():
            pltpu.sync_copy(zeros_hbm.at[pl.ds(r0, rows_pt)],
                            acc.at[pl.ds(r0, rows_pt)])

        def flush_acc(out):
            pltpu.sync_copy(acc.at[pl.ds(r0, rows_pt)],
                            out.at[pl.ds(r0, rows_pt), pl.ds(0, 16)])

        if with_deg:
            wid = c * NS + s

            def ar2(k, u):
                rb = wid * (ept2 // 128) + k * SUB
                return pltpu.make_async_copy(row2_hbm.at[pl.ds(rb, SUB)],
                                             rowv[u % 4], sem_ar[u % 4])

            def s2_copies(k, u):
                return [
                    pltpu.make_async_copy(val[1].at[pl.ds(0, 128)],
                                          acc.at[rowv[u % 4].at[j]],
                                          sem_s[u % 2])
                    for j in range(SUB)
                ]

            zero_acc()
            pltpu.sync_copy(ones_hbm, val[1].at[pl.ds(0, 128)])
            plsc.subcore_barrier()
            ar2(0, 0).start()
            ar2(1, 1).start()

            def deg_body(k0, carry):
                for u in range(4):
                    k = k0 * 4 + u

                    def drain(k=k, u=u):
                        for cp in s2_copies(k - 2, u + 2):
                            cp.wait()

                    if u < 2:
                        pl.when(k0 > 0)(drain)
                    else:
                        drain()

                    def prefetch(k=k, u=u):
                        ar2(k + 2, u + 2).start()

                    if u < 2:
                        prefetch()
                    else:
                        pl.when(k0 < kout2 - 1)(prefetch)
                    ar2(k, u).wait()
                    for cp in s2_copies(k, u):
                        cp.start(add=True)
                return carry

            lax.fori_loop(0, kout2, deg_body, 0)
            for cp in s2_copies(chunks2 - 2, 2):
                cp.wait()
            for cp in s2_copies(chunks2 - 1, 3):
                cp.wait()
            plsc.subcore_barrier()
            for core in range(NC):
                @pl.when(c == core)
                def _(core=core):
                    flush_acc(d_hbm[core])
            plsc.subcore_barrier()

        def run_group(tab, out, gidx):
            def ac(k, u):
                rb = s * (ept // 128) + k * SUB
                return pltpu.make_async_copy(col2_hbm.at[pl.ds(rb, SUB)],
                                             colv[u % 4], sem_ac[u % 4])

            def ar(k, u):
                rb = s * (ept // 128) + k * SUB
                return pltpu.make_async_copy(row2_hbm.at[pl.ds(rb, SUB)],
                                             rowv[u % 4], sem_ar[u % 4])

            def aw(k, u):
                eb = s * ept + k * CHUNK
                return pltpu.make_async_copy(wflat_hbm.at[pl.ds(eb, CHUNK)],
                                             wv[u % 4], sem_aw[u % 4])

            def g_copies(k, u):
                return [
                    pltpu.make_async_copy(tab.at[colv[u % 4].at[j]],
                                          gath[u % 4].at[pl.ds(j * 128, 128)],
                                          sem_g[u % 4])
                    for j in range(SUB)
                ]

            def s_copies(k, u):
                return [
                    pltpu.make_async_copy(val[u % 2].at[pl.ds(j * 128, 128)],
                                          acc.at[rowv[u % 4].at[j]],
                                          sem_s[u % 2])
                    for j in range(SUB)
                ]

            zero_acc()
            pltpu.sync_copy(b_hbm.at[gidx], bvec_s)
            plsc.subcore_barrier()

            for j in range(4):
                ac(j, j).start()
            for j in range(2):
                ar(j, j).start()
                aw(j, j).start()
            ac(0, 0).wait()
            for cp in g_copies(0, 0):
                cp.start()
            ac(1, 1).wait()
            for cp in g_copies(1, 1):
                cp.start()

            def outer_body(k0, carry):
                for u in range(4):
                    k = k0 * 4 + u

                    def drain_s(k=k, u=u):
                        for cp in s_copies(k - 2, u + 2):
                            cp.wait()

                    if u < 2:
                        pl.when(k0 > 0)(drain_s)
                    else:
                        drain_s()

                    def prefetch_rw(k=k, u=u):
                        ar(k + 2, u + 2).start()
                        aw(k + 2, u + 2).start()

                    if u < 2:
                        prefetch_rw()
                    else:
                        pl.when(k0 < kout - 1)(prefetch_rw)

                    for cp in g_copies(k, u):
                        cp.wait()

                    def prefetch_c(k=k, u=u):
                        ac(k + 4, u).start()

                    pl.when(k0 < kout - 1)(prefetch_c)

                    def start_g(k=k, u=u):
                        ac(k + 2, u + 2).wait()
                        for cp in g_copies(k + 2, u + 2):
                            cp.start()

                    if u < 2:
                        start_g()
                    else:
                        pl.when(k0 < kout - 1)(start_g)

                    aw(k, u).wait()
                    bvec = bvec_s[...]
                    gbuf = gath[u % 4]
                    vbuf = val[u % 2]
                    wbuf = wv[u % 4]

                    def grp_body(j, carry2):
                        wg = wbuf[pl.ds(j * L, L)]
                        for jj in range(L):
                            e = j * L + jj
                            wj = jnp.broadcast_to(
                                lax.slice(wg, (jj,), (jj + 1,)), (L,))
                            t = gbuf[e, :] * wj + bvec
                            vbuf[e, :] = jnp.maximum(t, 0.01 * t)
                        return carry2

                    lax.fori_loop(0, CHUNK // L, grp_body, 0)
                    ar(k, u).wait()
                    for cp in s_copies(k, u):
                        cp.start(add=True)
                return carry

            lax.fori_loop(0, kout, outer_body, 0)
            for cp in s_copies(chunks - 2, 2):
                cp.wait()
            for cp in s_copies(chunks - 1, 3):
                cp.wait()
            plsc.subcore_barrier()
            flush_acc(out)
            plsc.subcore_barrier()

        for core in range(NC):
            @pl.when(c == core)
            def _(core=core):
                for gl in range(gpc):
                    gidx = core * gpc + gl
                    run_group(tabs[gidx], outs[gidx], gidx)

    return pass_kernel


def kernel(x, edge, weight, W1, b1, W3, b3, W7, b7):
    n, _ = x.shape
    e = edge.shape[1]
    row = edge[0]
    col = edge[1]

    npad = ((n + 1 + 127) // 128) * 128
    estep = NC * NS * CHUNK
    epad = ((e + estep - 1) // estep) * estep
    pad = epad - e
    colp = jnp.concatenate([col, jnp.zeros((pad,), jnp.int32)])
    rowp = jnp.concatenate([row, jnp.full((pad,), n, jnp.int32)])
    wp = jnp.concatenate([weight, jnp.zeros((pad,), jnp.float32)])
    col2 = colp.reshape(-1, 128)
    row2 = rowp.reshape(-1, 128)
    zeros_hbm = jnp.zeros((npad, 16), jnp.float32)
    ones_hbm = jnp.ones((128, 16), jnp.float32)

    blk = 2000
    grid_n = n // blk
    wspec = pl.BlockSpec((blk, 128), lambda i: (i, 0))

    ya, yb, skip1 = pl.pallas_call(
        _stage_a_body,
        grid=(grid_n,),
        in_specs=[
            pl.BlockSpec((blk, x.shape[1]), lambda i: (i, 0)),
            pl.BlockSpec(W1.shape, lambda i: (0, 0)),
            pl.BlockSpec((1, 32), lambda i: (0, 0)),
        ],
        out_specs=[
            pl.BlockSpec((blk, 16), lambda i: (i, 0)),
            pl.BlockSpec((blk, 16), lambda i: (i, 0)),
            pl.BlockSpec((blk, 32), lambda i: (i, 0)),
        ],
        out_shape=[
            jax.ShapeDtypeStruct((n, 16), jnp.float32),
            jax.ShapeDtypeStruct((n, 16), jnp.float32),
            jax.ShapeDtypeStruct((n, 32), jnp.float32),
        ],
    )(x, W1, b1.reshape(1, 32))

    s1a, s1b, d0, d1 = _make_pass_kernel(2, npad, epad, True)(
        ya, yb, col2, row2, wp, b1.reshape(2, 16), zeros_hbm, ones_hbm)

    y20, y21, y22, y23, skip2 = pl.pallas_call(
        _stage_c_body,
        grid=(grid_n,),
        in_specs=[wspec, wspec, wspec, wspec,
                  pl.BlockSpec((blk, 32), lambda i: (i, 0)),
                  pl.BlockSpec(W3.shape, lambda i: (0, 0)),
                  pl.BlockSpec((1, 64), lambda i: (0, 0))],
        out_specs=[pl.BlockSpec((blk, 16), lambda i: (i, 0))] * 4
        + [pl.BlockSpec((blk, 64), lambda i: (i, 0))],
        out_shape=[jax.ShapeDtypeStruct((n, 16), jnp.float32)] * 4
        + [jax.ShapeDtypeStruct((n, 64), jnp.float32)],
    )(s1a, s1b, d0, d1, skip1, W3, b3.reshape(1, 64))

    s20, s21, s22, s23 = _make_pass_kernel(4, npad, epad, False)(
        y20, y21, y22, y23, col2, row2, wp, b3.reshape(4, 16), zeros_hbm)

    out = pl.pallas_call(
        functools.partial(_stage_e_body, n, grid_n),
        grid=(grid_n,),
        in_specs=[wspec, wspec, wspec, wspec, wspec, wspec,
                  pl.BlockSpec((blk, 64), lambda i: (i, 0)),
                  pl.BlockSpec(W7.shape, lambda i: (0, 0)),
                  pl.BlockSpec((1, 2), lambda i: (0, 0))],
        out_specs=pl.BlockSpec((1, 2), lambda i: (0, 0)),
        out_shape=jax.ShapeDtypeStruct((1, 2), jnp.float32),
        scratch_shapes=[pltpu.VMEM((8, 64), jnp.float32)],
    )(s20, s21, s22, s23, d0, d1, skip2, W7, b7.reshape(1, 2))
    return out
